# baseline (speedup 1.0000x reference)
import functools

import jax
import jax.numpy as jnp
from jax import lax
from jax.experimental import pallas as pl
from jax.experimental.pallas import tpu as pltpu
from jax.experimental.pallas import tpu_sc as plsc

NUM_BINS = 10
EMBED_DIM = 32
HP_CH = 10

NC = 2
NS = 16
L = 16
NW = NC * NS

CHUNK = 1024


def _sc_body(x_hbm, w_hbm, out_hbm, tbl, inbufs, outbufs, insems, outsems,
             *, n_ch, rows_per_w):
    wid = lax.axis_index("s") * NC + lax.axis_index("c")
    n_chunks = rows_per_w // CHUNK

    pltpu.sync_copy(w_hbm, tbl)

    lanes = lax.iota(jnp.int32, L)
    il_in = lanes * n_ch
    il_out = lanes * EMBED_DIM
    row0 = wid * rows_per_w

    def in_issue(chunk, buf):
        base = (row0 + chunk * CHUNK) * n_ch
        pltpu.async_copy(
            x_hbm.at[pl.ds(base, CHUNK * n_ch)], inbufs[buf], insems[buf])

    def in_wait(buf):
        pltpu.make_async_copy(
            x_hbm.at[pl.ds(0, CHUNK * n_ch)], inbufs[buf], insems[buf]).wait()

    def out_issue(chunk, buf):
        base = (row0 + chunk * CHUNK) * EMBED_DIM
        pltpu.async_copy(
            outbufs[buf], out_hbm.at[pl.ds(base, CHUNK * EMBED_DIM)],
            outsems[buf])

    def out_wait(buf):
        pltpu.make_async_copy(
            outbufs[buf], out_hbm.at[pl.ds(0, CHUNK * EMBED_DIM)],
            outsems[buf]).wait()

    def compute(inb, outb):
        def grp(g, carry):
            num = plsc.load_gather(inb, [il_in + (g * (L * n_ch) + HP_CH)])
            den = plsc.load_gather(inb, [il_in + (g * (L * n_ch) + HP_CH + 1)])
            idx = jnp.zeros((L,), jnp.int32)
            for k in range(NUM_BINS - 1):
                idx = idx + (den * (k / 8.0) < num).astype(jnp.int32)
            ob = il_out + g * (L * EMBED_DIM)
            for e in range(EMBED_DIM):
                vals = plsc.load_gather(tbl, [idx + e * NUM_BINS])
                plsc.store_scatter(outb, [ob + e], vals)
            return carry

        lax.fori_loop(0, CHUNK // L, grp, 0, unroll=False)

    in_issue(0, 0)

    def step(chunk, carry):
        for b in range(2):

            @pl.when(chunk % 2 == b)
            def _():
                in_wait(b)

                @pl.when(chunk + 1 < n_chunks)
                def _():
                    in_issue(chunk + 1, 1 - b)

                @pl.when(chunk >= 2)
                def _():
                    out_wait(b)

                compute(inbufs[b], outbufs[b])
                out_issue(chunk, b)

        return carry

    lax.fori_loop(0, n_chunks, step, 0)
    out_wait((n_chunks - 2) % 2)
    out_wait((n_chunks - 1) % 2)


def kernel(blstats, W):
    b, t, n_ch = blstats.shape
    n = b * t
    rows_per_w = n // NW

    x_flat = blstats.reshape(n * n_ch)
    w_flat = W.reshape(EMBED_DIM * NUM_BINS)

    mesh = plsc.VectorSubcoreMesh(core_axis_name="c", subcore_axis_name="s")
    body = functools.partial(_sc_body, n_ch=n_ch, rows_per_w=rows_per_w)
    sc_call = pl.kernel(
        body,
        out_type=jax.ShapeDtypeStruct((n * EMBED_DIM,), jnp.float32),
        mesh=mesh,
        compiler_params=pltpu.CompilerParams(
            use_tc_tiling_on_sc=False, needs_layout_passes=False),
        scratch_types=[
            pltpu.VMEM((NUM_BINS * EMBED_DIM,), jnp.float32),
            [pltpu.VMEM((CHUNK * n_ch,), jnp.float32) for _ in range(2)],
            [pltpu.VMEM((CHUNK * EMBED_DIM,), jnp.float32) for _ in range(2)],
            [pltpu.SemaphoreType.DMA for _ in range(2)],
            [pltpu.SemaphoreType.DMA for _ in range(2)],
        ],
    )
    out = sc_call(x_flat, w_flat)
    return out.reshape(b, t, EMBED_DIM)
